# Initial kernel scaffold; baseline (speedup 1.0000x reference)
import jax
import jax.numpy as jnp
from jax.experimental import pallas as pl


def kernel(m0, m1, m2, m3):
    raise NotImplementedError("write your pallas kernel here")


# speedup vs baseline: 3757.8756x; 3757.8756x over previous
import functools

import jax
import jax.numpy as jnp
from jax import lax
from jax.experimental import pallas as pl
from jax.experimental.pallas import tpu as pltpu
from jax.experimental.pallas import tpu_sc as plsc

BATCH = 16384
NS = 5
NFACT = 4
NRULES = NS ** NFACT

_info = plsc.get_sparse_core_info()
_NC, _NSUB, _L = _info.num_cores, _info.num_subcores, _info.num_lanes
NW = _NC * _NSUB
ROWS_PER_W = BATCH // NW
RB = 16
NBLK = ROWS_PER_W // RB
MT_W = NFACT * NS * ROWS_PER_W
BUF_W = RB * NRULES


def _sc_call(mt):
    mesh = plsc.VectorSubcoreMesh(core_axis_name="c", subcore_axis_name="s")

    @functools.partial(
        pl.kernel,
        mesh=mesh,
        out_type=jax.ShapeDtypeStruct((BATCH * NRULES,), jnp.float32),
        compiler_params=pltpu.CompilerParams(needs_layout_passes=False),
        scratch_types=[
            pltpu.VMEM((MT_W,), jnp.float32),
            pltpu.VMEM((BUF_W,), jnp.float32),
        ],
    )
    def k(mt_hbm, out_hbm, mt_v, buf_v):
        wid = lax.axis_index("s") * _NC + lax.axis_index("c")
        pltpu.sync_copy(mt_hbm.at[pl.ds(wid * MT_W, MT_W)], mt_v)
        lane_off = lax.iota(jnp.int32, _L) * NRULES

        def block(t, carry):
            vs = [[mt_v[pl.ds((j * NS + i) * ROWS_PER_W + t * RB, RB)]
                   for i in range(NS)] for j in range(NFACT)]
            for i0 in range(NS):
                v0 = vs[0][i0]
                for i1 in range(NS):
                    v01 = v0 * vs[1][i1]
                    for i2 in range(NS):
                        v012 = v01 * vs[2][i2]
                        for i3 in range(NS):
                            r = ((i0 * NS + i1) * NS + i2) * NS + i3
                            val = v012 * vs[3][i3]
                            plsc.store_scatter(buf_v, [lane_off + r], val)
            out_off = (wid * ROWS_PER_W + t * RB) * NRULES
            pltpu.sync_copy(buf_v, out_hbm.at[pl.ds(out_off, BUF_W)])
            return carry

        lax.fori_loop(0, NBLK, block, 0)

    return k(mt)


def kernel(m0, m1, m2, m3):
    mt = jnp.concatenate([m0.T, m1.T, m2.T, m3.T], axis=0)
    mt = mt.reshape(NFACT * NS, NW, ROWS_PER_W).transpose(1, 0, 2)
    return _sc_call(mt.reshape(-1)).reshape(BATCH, NRULES)


# speedup vs baseline: 4173.8869x; 1.1107x over previous
import functools

import jax
import jax.numpy as jnp
from jax import lax
from jax.experimental import pallas as pl
from jax.experimental.pallas import tpu as pltpu
from jax.experimental.pallas import tpu_sc as plsc

BATCH = 16384
NS = 5
NFACT = 4
NRULES = NS ** NFACT

_info = plsc.get_sparse_core_info()
_NC, _NSUB, _L = _info.num_cores, _info.num_subcores, _info.num_lanes
NW = _NC * _NSUB
ROWS_PER_W = BATCH // NW
RB = 16
NBLK = ROWS_PER_W // RB
MT_W = NFACT * NS * ROWS_PER_W
BUF_W = RB * NRULES


def _sc_call(mt):
    mesh = plsc.VectorSubcoreMesh(core_axis_name="c", subcore_axis_name="s")

    @functools.partial(
        pl.kernel,
        mesh=mesh,
        out_type=jax.ShapeDtypeStruct((BATCH * NRULES,), jnp.float32),
        compiler_params=pltpu.CompilerParams(needs_layout_passes=False),
        scratch_types=[
            pltpu.VMEM((MT_W,), jnp.float32),
            pltpu.VMEM((2 * BUF_W,), jnp.float32),
            pltpu.SemaphoreType.DMA,
            pltpu.SemaphoreType.DMA,
        ],
    )
    def k(mt_hbm, out_hbm, mt_v, buf_v, sem0, sem1):
        wid = lax.axis_index("s") * _NC + lax.axis_index("c")
        pltpu.sync_copy(mt_hbm.at[pl.ds(wid * MT_W, MT_W)], mt_v)
        lane_off = lax.iota(jnp.int32, _L) * NRULES

        def _drain(sem):
            pltpu.make_async_copy(buf_v.at[pl.ds(0, BUF_W)],
                                  out_hbm.at[pl.ds(0, BUF_W)], sem).wait()

        def block(t, carry):
            par = jnp.bitwise_and(t, 1)
            base_idx = lane_off + par * BUF_W

            @pl.when(t >= 2)
            def _():
                @pl.when(par == 0)
                def _():
                    _drain(sem0)
                @pl.when(par == 1)
                def _():
                    _drain(sem1)

            vs = [[mt_v[pl.ds((j * NS + i) * ROWS_PER_W + t * RB, RB)]
                   for i in range(NS)] for j in range(NFACT)]
            for i0 in range(NS):
                v0 = vs[0][i0]
                for i1 in range(NS):
                    v01 = v0 * vs[1][i1]
                    for i2 in range(NS):
                        v012 = v01 * vs[2][i2]
                        for i3 in range(NS):
                            r = ((i0 * NS + i1) * NS + i2) * NS + i3
                            val = v012 * vs[3][i3]
                            plsc.store_scatter(buf_v, [base_idx + r], val)
            out_off = (wid * ROWS_PER_W + t * RB) * NRULES

            @pl.when(par == 0)
            def _():
                pltpu.async_copy(buf_v.at[pl.ds(0, BUF_W)],
                                 out_hbm.at[pl.ds(out_off, BUF_W)], sem0)

            @pl.when(par == 1)
            def _():
                pltpu.async_copy(buf_v.at[pl.ds(BUF_W, BUF_W)],
                                 out_hbm.at[pl.ds(out_off, BUF_W)], sem1)
            return carry

        lax.fori_loop(0, NBLK, block, 0)
        _drain(sem0)
        _drain(sem1)

    return k(mt)


def kernel(m0, m1, m2, m3):
    mt = jnp.concatenate([m0.T, m1.T, m2.T, m3.T], axis=0)
    mt = mt.reshape(NFACT * NS, NW, ROWS_PER_W).transpose(1, 0, 2)
    return _sc_call(mt.reshape(-1)).reshape(BATCH, NRULES)
